# baseline (speedup 1.0000x reference)
import functools

import jax
import jax.numpy as jnp
import numpy as np
from jax import lax
from jax.experimental import pallas as pl
from jax.experimental.pallas import tpu as pltpu
from jax.experimental.pallas import tpu_sc as plsc

N = 10000
E = 320000
D = 128
H = 8
DH = D // H

NC = 2
NS = 16
NW = NC * NS
NPAD = 10240

SCW = 128
GC = 80
EB = 1000
NB = 2000
EBN = 1000

_F32 = jnp.float32


def _pipe2(n_iter, start_fn, wait_fn, consume_fn):
    start_fn(0, 0)
    start_fn(1, 1)

    def outer(g, _):
        for b in range(2):
            j = 2 * g + b
            wait_fn(j, b)
            consume_fn(j, b)

            @pl.when(j + 2 < n_iter)
            def _():
                start_fn(j + 2, b)

        return 0

    lax.fori_loop(0, (n_iter - 1) // 2, outer, 0, unroll=False)
    j = n_iter - 1
    wait_fn(j, j % 2)
    consume_fn(j, j % 2)


def _gather_body(table, idx3d_hbm, out, idx_v, buf0, buf1, sem0, sem1,
                 *, width, rows_per_w):
    c = lax.axis_index("c")
    s = lax.axis_index("s")
    wid = s * NC + c
    n_iter = rows_per_w // GC
    pltpu.sync_copy(idx3d_hbm.at[wid], idx_v)
    bufs = (buf0, buf1)
    sems = (sem0, sem1)

    def start(j, b):
        pltpu.async_copy(table.at[idx_v.at[j]], bufs[b], sems[b])

    def wait(j, b):
        pltpu.make_async_copy(table.at[idx_v.at[j]], bufs[b], sems[b]).wait()

    def consume(j, b):
        off = wid * rows_per_w + j * GC
        pltpu.sync_copy(bufs[b], out.at[pl.ds(off, GC)])

    _pipe2(n_iter, start, wait, consume)


def _make_gather(width):
    rows_per_w = E // NW
    mesh = plsc.VectorSubcoreMesh(core_axis_name="c", subcore_axis_name="s")
    return pl.kernel(
        functools.partial(_gather_body, width=width, rows_per_w=rows_per_w),
        out_type=jax.ShapeDtypeStruct((E, width), _F32),
        mesh=mesh,
        scratch_types=[
            pltpu.VMEM((rows_per_w // GC, GC), jnp.int32),
            pltpu.VMEM((GC, width), _F32),
            pltpu.VMEM((GC, width), _F32),
            pltpu.SemaphoreType.DMA,
            pltpu.SemaphoreType.DMA,
        ],
    )


def _zero_buf(b128):
    zv = jnp.zeros((16,), _F32)
    width = b128.shape[1]

    def zrow(r, _):
        for kk in range(width // 16):
            b128[r, pl.ds(kk * 16, 16)] = zv
        return 0

    lax.fori_loop(0, GC, zrow, 0, unroll=False)


def _zero_table(t128, b128, r0, nchunk):
    def zcp(k, _):
        pltpu.sync_copy(b128, t128.at[pl.ds(r0 + k * GC, GC)])
        return 0

    lax.fori_loop(0, nchunk, zcp, 0, unroll=False)


def _writeback(t128, b128, out_slice_fn, r0, nchunk):
    def wb(k, _):
        rr = r0 + k * GC
        pltpu.sync_copy(t128.at[pl.ds(rr, GC)], b128)
        pltpu.sync_copy(b128, out_slice_fn(rr))
        return 0

    lax.fori_loop(0, nchunk, wb, 0, unroll=False)


def _scatter2_body(c2_hbm, idx4d_hbm, acc2_out, t128, idx_v, b0, b1,
                   sem0, sem1):
    c = lax.axis_index("c")
    s = lax.axis_index("s")
    nrow = NPAD // NS
    r0 = s * nrow
    _zero_buf(b0)
    _zero_table(t128, b0, r0, nrow // GC)
    plsc.subcore_barrier()

    bufs = (b0, b1)
    sems = (sem0, sem1)
    rows_per_t = E // NS
    n_half = (rows_per_t // GC) // 2
    for half in range(2):
        pltpu.sync_copy(idx4d_hbm.at[s, half], idx_v)

        def start(j, b, half=half):
            off = s * rows_per_t + (half * n_half + j) * GC
            pltpu.async_copy(c2_hbm.at[c, pl.ds(off, GC)], bufs[b], sems[b])

        def wait(j, b, half=half):
            off = s * rows_per_t + (half * n_half + j) * GC
            pltpu.make_async_copy(c2_hbm.at[c, pl.ds(off, GC)], bufs[b],
                                  sems[b]).wait()

        def consume(j, b):
            pltpu.sync_copy(bufs[b], t128.at[idx_v.at[j]], add=True)

        _pipe2(n_half, start, wait, consume)
    plsc.subcore_barrier()
    _writeback(t128, b0, lambda rr: acc2_out.at[c, pl.ds(rr, GC)],
               r0, nrow // GC)


def _make_scatter2():
    mesh = plsc.VectorSubcoreMesh(core_axis_name="c", subcore_axis_name="s")
    n_half = ((E // NS) // GC) // 2
    return pl.kernel(
        _scatter2_body,
        out_type=jax.ShapeDtypeStruct((2, NPAD, D), _F32),
        mesh=mesh,
        scratch_types=[
            pltpu.MemorySpace.VMEM_SHARED((NPAD, D), _F32),
            pltpu.VMEM((n_half, GC), jnp.int32),
            pltpu.VMEM((GC, D), _F32),
            pltpu.VMEM((GC, D), _F32),
            pltpu.SemaphoreType.DMA,
            pltpu.SemaphoreType.DMA,
        ],
    )


def _scatter_ss_body(sc_hbm, idx4d_hbm, ss2_out, t128, idx_v, b0, b1,
                     sem0, sem1):
    c = lax.axis_index("c")
    s = lax.axis_index("s")
    nrow = NPAD // NS
    r0 = s * nrow
    _zero_buf(b0)
    _zero_table(t128, b0, r0, nrow // GC)
    plsc.subcore_barrier()

    bufs = (b0, b1)
    sems = (sem0, sem1)
    rows_per_t = (E // 2) // NS
    n_iter = rows_per_t // GC
    pltpu.sync_copy(idx4d_hbm.at[c, s], idx_v)

    def start(j, b):
        off = c * (E // 2) + s * rows_per_t + j * GC
        pltpu.async_copy(sc_hbm.at[pl.ds(off, GC)], bufs[b], sems[b])

    def wait(j, b):
        off = c * (E // 2) + s * rows_per_t + j * GC
        pltpu.make_async_copy(sc_hbm.at[pl.ds(off, GC)], bufs[b],
                              sems[b]).wait()

    def consume(j, b):
        pltpu.sync_copy(bufs[b], t128.at[idx_v.at[j]], add=True)

    _pipe2(n_iter, start, wait, consume)
    plsc.subcore_barrier()
    _writeback(t128, b0, lambda rr: ss2_out.at[c, pl.ds(rr, GC)],
               r0, nrow // GC)


def _make_scatter_ss():
    mesh = plsc.VectorSubcoreMesh(core_axis_name="c", subcore_axis_name="s")
    n_iter = ((E // 2) // NS) // GC
    return pl.kernel(
        _scatter_ss_body,
        out_type=jax.ShapeDtypeStruct((2, NPAD, SCW), _F32),
        mesh=mesh,
        scratch_types=[
            pltpu.MemorySpace.VMEM_SHARED((NPAD, SCW), _F32),
            pltpu.VMEM((n_iter, GC), jnp.int32),
            pltpu.VMEM((GC, SCW), _F32),
            pltpu.VMEM((GC, SCW), _F32),
            pltpu.SemaphoreType.DMA,
            pltpu.SemaphoreType.DMA,
        ],
    )


def _nodeproj_body(x_ref, wkv_ref, bkv_ref, wq_ref, bq_ref, kv_ref, q_ref):
    x = x_ref[...]
    kv_ref[...] = jnp.dot(x, wkv_ref[...], preferred_element_type=_F32) + bkv_ref[...]
    q_ref[...] = jnp.dot(x, wq_ref[...], preferred_element_type=_F32) + bq_ref[...]


def _edge_body(ea_ref, skv_ref, dq_ref, eww_ref, ewb_ref, ebw_ref, ebb_ref,
               oew_ref, oeb_ref, aw128_ref, awsc_ref,
               ep_ref, c2_ref, sc_ref, stats_ref, acc_ref):
    i = pl.program_id(0)
    ea = ea_ref[...]
    skv = skv_ref[...]
    k = skv[:, :D]
    v = skv[:, D:]
    s0 = k + dq_ref[...]
    ew = jnp.dot(ea, eww_ref[...], preferred_element_type=_F32) + ebw_ref[...]
    eb = jnp.dot(ea, ewb_ref[...], preferred_element_type=_F32) + ebb_ref[...]
    sc = s0 * ew
    sc = jnp.sqrt(jnp.maximum(sc, 0.0)) - jnp.sqrt(jnp.maximum(-sc, 0.0))
    et = jnp.maximum(sc + eb, 0.0)
    ep = jnp.dot(et, oew_ref[...], preferred_element_type=_F32) + oeb_ref[...] + ea
    ep_ref[...] = ep.astype(jnp.bfloat16)
    s128 = jnp.clip(jnp.dot(et, aw128_ref[...], preferred_element_type=_F32), -5.0, 5.0)
    p128 = jnp.exp(s128 - 5.0)
    if SCW == D:
        sc_ref[...] = p128
    else:
        ssc = jnp.clip(jnp.dot(et, awsc_ref[...], preferred_element_type=_F32),
                       -5.0, 5.0)
        sc_ref[...] = jnp.exp(ssc - 5.0)
    c2_ref[0] = p128 * v
    c2_ref[1] = p128 * et

    @pl.when(i == 0)
    def _():
        acc_ref[...] = jnp.zeros_like(acc_ref)

    acc_ref[0:1] = acc_ref[0:1] + jnp.sum(ep, axis=0, keepdims=True)
    acc_ref[1:2] = acc_ref[1:2] + jnp.sum(ep * ep, axis=0, keepdims=True)

    @pl.when(i == pl.num_programs(0) - 1)
    def _():
        stats_ref[...] = acc_ref[...]


def _bn_apply(x, stats_ref, g_ref, b_ref, count):
    mu = stats_ref[0:1] * (1.0 / count)
    var = stats_ref[1:2] * (1.0 / count) - mu * mu
    inv = lax.rsqrt(var + 1e-5)
    return g_ref[...] * (x - mu) * inv + b_ref[...]


def _bne_body(ep_ref, stats_ref, g_ref, b_ref, out_ref):
    out_ref[...] = _bn_apply(ep_ref[...].astype(_F32), stats_ref, g_ref, b_ref, E)


def _f1_body(wv_ref, rv_ref, ssa_ref, ssb_ref, x_ref, ld_ref, exp_ref, vebd_ref,
             dc0_ref, dc1_ref, ohw_ref, ohb_ref, hpre_ref, stats_ref, acc_ref):
    i = pl.program_id(0)
    ssr = jnp.dot(ssa_ref[0] + ssb_ref[0], exp_ref[...],
                  preferred_element_type=_F32)
    inv = 1.0 / (ssr + 1e-16)
    wv = wv_ref[0] * inv
    rv = jnp.dot(rv_ref[0] * inv, vebd_ref[...], preferred_element_type=_F32)
    wv = wv + rv
    hc = wv * (dc0_ref[...] + ld_ref[...] * dc1_ref[...])
    hp = jnp.dot(hc, ohw_ref[...], preferred_element_type=_F32) + ohb_ref[...] + x_ref[...]
    hpre_ref[...] = hp

    @pl.when(i == 0)
    def _():
        acc_ref[...] = jnp.zeros_like(acc_ref)

    acc_ref[0:1] = acc_ref[0:1] + jnp.sum(hp, axis=0, keepdims=True)
    acc_ref[1:2] = acc_ref[1:2] + jnp.sum(hp * hp, axis=0, keepdims=True)

    @pl.when(i == pl.num_programs(0) - 1)
    def _():
        stats_ref[...] = acc_ref[...]


def _f2_body(hp_ref, stats_ref, g1_ref, b1_ref, f1w_ref, f1b_ref,
             f2w_ref, f2b_ref, t_ref, stats2_ref, acc_ref):
    i = pl.program_id(0)
    h1 = _bn_apply(hp_ref[...], stats_ref, g1_ref, b1_ref, N)
    ffn = jnp.maximum(jnp.dot(h1, f1w_ref[...], preferred_element_type=_F32) + f1b_ref[...], 0.0)
    t = h1 + jnp.dot(ffn, f2w_ref[...], preferred_element_type=_F32) + f2b_ref[...]
    t_ref[...] = t

    @pl.when(i == 0)
    def _():
        acc_ref[...] = jnp.zeros_like(acc_ref)

    acc_ref[0:1] = acc_ref[0:1] + jnp.sum(t, axis=0, keepdims=True)
    acc_ref[1:2] = acc_ref[1:2] + jnp.sum(t * t, axis=0, keepdims=True)

    @pl.when(i == pl.num_programs(0) - 1)
    def _():
        stats2_ref[...] = acc_ref[...]


def _f3_body(t_ref, stats_ref, g_ref, b_ref, out_ref):
    out_ref[...] = _bn_apply(t_ref[...], stats_ref, g_ref, b_ref, N)


def _row(v):
    return v.reshape(1, -1)


def _wspec(shape):
    nd = len(shape)
    return pl.BlockSpec(shape, lambda i: (0,) * nd)


_ARB = pltpu.CompilerParams(dimension_semantics=("arbitrary",))


def kernel(x, edge_index, edge_attr, log_deg, num_nodes, Qw, Qb, Kw, Kb, Ew, Eb,
           Vw, Vb, Aw, VeRow, Ohw, Ohb, Oew, Oeb, deg_coef, bn1h_g, bn1h_b,
           bn1e_g, bn1e_b, bn2h_g, bn2h_b, f1w, f1b, f2w, f2b):
    perm = np.arange(D)
    hh, dd = perm // DH, perm % DH
    perm_w = jnp.asarray(hh * 2 * DH + dd)
    perm_b = jnp.asarray(hh * 2 * DH + DH + dd)
    Ew_w = Ew[:, perm_w]
    Ew_b = Ew[:, perm_b]
    eb_w = _row(Eb[perm_w])
    eb_b = _row(Eb[perm_b])
    A2 = Aw[:, :, 0]
    aw128 = jax.scipy.linalg.block_diag(
        *[jnp.broadcast_to(A2[:, h:h + 1], (DH, DH)) for h in range(H)])
    awsc = jax.scipy.linalg.block_diag(
        *[jnp.broadcast_to(A2[:, h:h + 1], (DH, SCW // H)) for h in range(H)])
    expm = jnp.asarray(np.kron(np.eye(H, dtype=np.float32),
                               np.full((SCW // H, DH), 1.0 / (SCW // H),
                                       np.float32)))
    vebd = jax.scipy.linalg.block_diag(*[VeRow[:, h, :] for h in range(H)])
    dc0 = _row(deg_coef[0, :, 0])
    dc1 = _row(deg_coef[0, :, 1])
    wkv = jnp.concatenate([Kw, Vw], axis=1)
    bkv = _row(jnp.concatenate([Kb, Vb]))
    src3w = edge_index[0].reshape(NW, (E // NW) // GC, GC)
    dst3w = edge_index[1].reshape(NW, (E // NW) // GC, GC)
    dst4t = edge_index[1].reshape(NS, 2, ((E // NS) // GC) // 2, GC)
    dst4c = edge_index[1].reshape(2, NS, ((E // 2) // NS) // GC, GC)
    ld128 = jnp.broadcast_to(log_deg, (N, D))

    kv_tab, q_tab = pl.pallas_call(
        _nodeproj_body,
        grid=(N // NB,),
        in_specs=[
            pl.BlockSpec((NB, D), lambda i: (i, 0)),
            _wspec((D, 2 * D)), _wspec((1, 2 * D)),
            _wspec((D, D)), _wspec((1, D)),
        ],
        out_specs=[
            pl.BlockSpec((NB, 2 * D), lambda i: (i, 0)),
            pl.BlockSpec((NB, D), lambda i: (i, 0)),
        ],
        out_shape=[
            jax.ShapeDtypeStruct((N, 2 * D), _F32),
            jax.ShapeDtypeStruct((N, D), _F32),
        ],
        compiler_params=_ARB,
    )(x, wkv, bkv, Qw, _row(Qb))

    skv = _make_gather(2 * D)(kv_tab, src3w)
    dq = _make_gather(D)(q_tab, dst3w)

    ep, c2, scw, stats_e = pl.pallas_call(
        _edge_body,
        grid=(E // EB,),
        in_specs=[
            pl.BlockSpec((EB, D), lambda i: (i, 0)),
            pl.BlockSpec((EB, 2 * D), lambda i: (i, 0)),
            pl.BlockSpec((EB, D), lambda i: (i, 0)),
            _wspec((D, D)), _wspec((D, D)), _wspec((1, D)), _wspec((1, D)),
            _wspec((D, D)), _wspec((1, D)), _wspec((D, D)), _wspec((D, SCW)),
        ],
        out_specs=[
            pl.BlockSpec((EB, D), lambda i: (i, 0)),
            pl.BlockSpec((2, EB, D), lambda i: (0, i, 0)),
            pl.BlockSpec((EB, SCW), lambda i: (i, 0)),
            pl.BlockSpec((8, D), lambda i: (0, 0)),
        ],
        out_shape=[
            jax.ShapeDtypeStruct((E, D), jnp.bfloat16),
            jax.ShapeDtypeStruct((2, E, D), _F32),
            jax.ShapeDtypeStruct((E, SCW), _F32),
            jax.ShapeDtypeStruct((8, D), _F32),
        ],
        scratch_shapes=[pltpu.VMEM((8, D), _F32)],
        compiler_params=_ARB,
    )(edge_attr, skv, dq, Ew_w, Ew_b, eb_w, eb_b, Oew, _row(Oeb), aw128, awsc)

    acc2 = _make_scatter2()(c2, dst4t)
    ss2 = _make_scatter_ss()(scw, dst4c)

    e_out = pl.pallas_call(
        _bne_body,
        grid=(E // EBN,),
        in_specs=[
            pl.BlockSpec((EBN, D), lambda i: (i, 0)),
            _wspec((8, D)), _wspec((1, D)), _wspec((1, D)),
        ],
        out_specs=pl.BlockSpec((EBN, D), lambda i: (i, 0)),
        out_shape=jax.ShapeDtypeStruct((E, D), _F32),
        compiler_params=_ARB,
    )(ep, stats_e, _row(bn1e_g), _row(bn1e_b))

    hpre, stats1 = pl.pallas_call(
        _f1_body,
        grid=(N // NB,),
        in_specs=[
            pl.BlockSpec((1, NB, D), lambda i: (0, i, 0)),
            pl.BlockSpec((1, NB, D), lambda i: (1, i, 0)),
            pl.BlockSpec((1, NB, SCW), lambda i: (0, i, 0)),
            pl.BlockSpec((1, NB, SCW), lambda i: (1, i, 0)),
            pl.BlockSpec((NB, D), lambda i: (i, 0)),
            pl.BlockSpec((NB, D), lambda i: (i, 0)),
            _wspec((SCW, D)), _wspec((D, D)),
            _wspec((1, D)), _wspec((1, D)), _wspec((D, D)), _wspec((1, D)),
        ],
        out_specs=[
            pl.BlockSpec((NB, D), lambda i: (i, 0)),
            pl.BlockSpec((8, D), lambda i: (0, 0)),
        ],
        out_shape=[
            jax.ShapeDtypeStruct((N, D), _F32),
            jax.ShapeDtypeStruct((8, D), _F32),
        ],
        scratch_shapes=[pltpu.VMEM((8, D), _F32)],
        compiler_params=_ARB,
    )(acc2, acc2, ss2, ss2, x, ld128, expm, vebd, dc0, dc1, Ohw, _row(Ohb))

    t_arr, stats2 = pl.pallas_call(
        _f2_body,
        grid=(N // NB,),
        in_specs=[
            pl.BlockSpec((NB, D), lambda i: (i, 0)),
            _wspec((8, D)), _wspec((1, D)), _wspec((1, D)),
            _wspec((D, 2 * D)), _wspec((1, 2 * D)),
            _wspec((2 * D, D)), _wspec((1, D)),
        ],
        out_specs=[
            pl.BlockSpec((NB, D), lambda i: (i, 0)),
            pl.BlockSpec((8, D), lambda i: (0, 0)),
        ],
        out_shape=[
            jax.ShapeDtypeStruct((N, D), _F32),
            jax.ShapeDtypeStruct((8, D), _F32),
        ],
        scratch_shapes=[pltpu.VMEM((8, D), _F32)],
        compiler_params=_ARB,
    )(hpre, stats1, _row(bn1h_g), _row(bn1h_b), f1w, _row(f1b), f2w, _row(f2b))

    h_out = pl.pallas_call(
        _f3_body,
        grid=(N // NB,),
        in_specs=[
            pl.BlockSpec((NB, D), lambda i: (i, 0)),
            _wspec((8, D)), _wspec((1, D)), _wspec((1, D)),
        ],
        out_specs=pl.BlockSpec((NB, D), lambda i: (i, 0)),
        out_shape=jax.ShapeDtypeStruct((N, D), _F32),
        compiler_params=_ARB,
    )(t_arr, stats2, _row(bn2h_g), _row(bn2h_b))

    return (h_out, e_out)


# speedup vs baseline: 1.0944x; 1.0944x over previous
import functools

import jax
import jax.numpy as jnp
import numpy as np
from jax import lax
from jax.experimental import pallas as pl
from jax.experimental.pallas import tpu as pltpu
from jax.experimental.pallas import tpu_sc as plsc

N = 10000
E = 320000
D = 128
H = 8
DH = D // H

NC = 2
NS = 16
NW = NC * NS
NPAD = 10240

SCW = 128
GC = 80
EB = 2000
NB = 2000
EBN = 2000

_F32 = jnp.float32


def _pipe2(n_iter, start_fn, wait_fn, consume_fn):
    start_fn(0, 0)
    start_fn(1, 1)

    def outer(g, _):
        for b in range(2):
            j = 2 * g + b
            wait_fn(j, b)
            consume_fn(j, b)

            @pl.when(j + 2 < n_iter)
            def _():
                start_fn(j + 2, b)

        return 0

    lax.fori_loop(0, (n_iter - 1) // 2, outer, 0, unroll=False)
    j = n_iter - 1
    wait_fn(j, j % 2)
    consume_fn(j, j % 2)


def _gather_body(table, idx3d_hbm, out, idx_v, buf0, buf1, sem0, sem1,
                 *, width, rows_per_w):
    c = lax.axis_index("c")
    s = lax.axis_index("s")
    wid = s * NC + c
    n_iter = rows_per_w // GC
    pltpu.sync_copy(idx3d_hbm.at[wid], idx_v)
    bufs = (buf0, buf1)
    sems = (sem0, sem1)

    def start(j, b):
        pltpu.async_copy(table.at[idx_v.at[j]], bufs[b], sems[b])

    def wait(j, b):
        pltpu.make_async_copy(table.at[idx_v.at[j]], bufs[b], sems[b]).wait()

    def consume(j, b):
        off = wid * rows_per_w + j * GC
        pltpu.sync_copy(bufs[b], out.at[pl.ds(off, GC)])

    _pipe2(n_iter, start, wait, consume)


def _make_gather(width):
    rows_per_w = E // NW
    mesh = plsc.VectorSubcoreMesh(core_axis_name="c", subcore_axis_name="s")
    return pl.kernel(
        functools.partial(_gather_body, width=width, rows_per_w=rows_per_w),
        out_type=jax.ShapeDtypeStruct((E, width), _F32),
        mesh=mesh,
        scratch_types=[
            pltpu.VMEM((rows_per_w // GC, GC), jnp.int32),
            pltpu.VMEM((GC, width), _F32),
            pltpu.VMEM((GC, width), _F32),
            pltpu.SemaphoreType.DMA,
            pltpu.SemaphoreType.DMA,
        ],
    )


def _zero_buf(b128):
    zv = jnp.zeros((16,), _F32)
    width = b128.shape[1]

    def zrow(r, _):
        for kk in range(width // 16):
            b128[r, pl.ds(kk * 16, 16)] = zv
        return 0

    lax.fori_loop(0, GC, zrow, 0, unroll=False)


def _zero_table(t128, b128, r0, nchunk):
    def zcp(k, _):
        pltpu.sync_copy(b128, t128.at[pl.ds(r0 + k * GC, GC)])
        return 0

    lax.fori_loop(0, nchunk, zcp, 0, unroll=False)


def _writeback(t128, b128, out_slice_fn, r0, nchunk):
    def wb(k, _):
        rr = r0 + k * GC
        pltpu.sync_copy(t128.at[pl.ds(rr, GC)], b128)
        pltpu.sync_copy(b128, out_slice_fn(rr))
        return 0

    lax.fori_loop(0, nchunk, wb, 0, unroll=False)


def _scatter2_body(c2_hbm, idx4d_hbm, acc2_out, t128, idx_v, b0, b1,
                   sem0, sem1):
    c = lax.axis_index("c")
    s = lax.axis_index("s")
    nrow = NPAD // NS
    r0 = s * nrow
    _zero_buf(b0)
    _zero_table(t128, b0, r0, nrow // GC)
    plsc.subcore_barrier()

    bufs = (b0, b1)
    sems = (sem0, sem1)
    rows_per_t = E // NS
    n_half = (rows_per_t // GC) // 2
    for half in range(2):
        pltpu.sync_copy(idx4d_hbm.at[s, half], idx_v)

        def start(j, b, half=half):
            off = s * rows_per_t + (half * n_half + j) * GC
            pltpu.async_copy(c2_hbm.at[c, pl.ds(off, GC)], bufs[b], sems[b])

        def wait(j, b, half=half):
            off = s * rows_per_t + (half * n_half + j) * GC
            pltpu.make_async_copy(c2_hbm.at[c, pl.ds(off, GC)], bufs[b],
                                  sems[b]).wait()

        def consume(j, b):
            pltpu.sync_copy(bufs[b], t128.at[idx_v.at[j]], add=True)

        _pipe2(n_half, start, wait, consume)
    plsc.subcore_barrier()
    _writeback(t128, b0, lambda rr: acc2_out.at[c, pl.ds(rr, GC)],
               r0, nrow // GC)


def _make_scatter2():
    mesh = plsc.VectorSubcoreMesh(core_axis_name="c", subcore_axis_name="s")
    n_half = ((E // NS) // GC) // 2
    return pl.kernel(
        _scatter2_body,
        out_type=jax.ShapeDtypeStruct((2, NPAD, D), _F32),
        mesh=mesh,
        scratch_types=[
            pltpu.MemorySpace.VMEM_SHARED((NPAD, D), _F32),
            pltpu.VMEM((n_half, GC), jnp.int32),
            pltpu.VMEM((GC, D), _F32),
            pltpu.VMEM((GC, D), _F32),
            pltpu.SemaphoreType.DMA,
            pltpu.SemaphoreType.DMA,
        ],
    )


def _scatter_ss_body(sc_hbm, idx4d_hbm, ss2_out, t128, idx_v, b0, b1,
                     sem0, sem1):
    c = lax.axis_index("c")
    s = lax.axis_index("s")
    nrow = NPAD // NS
    r0 = s * nrow
    _zero_buf(b0)
    _zero_table(t128, b0, r0, nrow // GC)
    plsc.subcore_barrier()

    bufs = (b0, b1)
    sems = (sem0, sem1)
    rows_per_t = (E // 2) // NS
    n_iter = rows_per_t // GC
    pltpu.sync_copy(idx4d_hbm.at[c, s], idx_v)

    def start(j, b):
        off = c * (E // 2) + s * rows_per_t + j * GC
        pltpu.async_copy(sc_hbm.at[pl.ds(off, GC)], bufs[b], sems[b])

    def wait(j, b):
        off = c * (E // 2) + s * rows_per_t + j * GC
        pltpu.make_async_copy(sc_hbm.at[pl.ds(off, GC)], bufs[b],
                              sems[b]).wait()

    def consume(j, b):
        pltpu.sync_copy(bufs[b], t128.at[idx_v.at[j]], add=True)

    _pipe2(n_iter, start, wait, consume)
    plsc.subcore_barrier()
    _writeback(t128, b0, lambda rr: ss2_out.at[c, pl.ds(rr, GC)],
               r0, nrow // GC)


def _make_scatter_ss():
    mesh = plsc.VectorSubcoreMesh(core_axis_name="c", subcore_axis_name="s")
    n_iter = ((E // 2) // NS) // GC
    return pl.kernel(
        _scatter_ss_body,
        out_type=jax.ShapeDtypeStruct((2, NPAD, SCW), _F32),
        mesh=mesh,
        scratch_types=[
            pltpu.MemorySpace.VMEM_SHARED((NPAD, SCW), _F32),
            pltpu.VMEM((n_iter, GC), jnp.int32),
            pltpu.VMEM((GC, SCW), _F32),
            pltpu.VMEM((GC, SCW), _F32),
            pltpu.SemaphoreType.DMA,
            pltpu.SemaphoreType.DMA,
        ],
    )


def _nodeproj_body(x_ref, wkv_ref, bkv_ref, wq_ref, bq_ref, kv_ref, q_ref):
    x = x_ref[...]
    kv_ref[...] = jnp.dot(x, wkv_ref[...], preferred_element_type=_F32) + bkv_ref[...]
    q_ref[...] = jnp.dot(x, wq_ref[...], preferred_element_type=_F32) + bq_ref[...]


def _edge_body(ea_ref, skv_ref, dq_ref, eww_ref, ewb_ref, ebw_ref, ebb_ref,
               oew_ref, oeb_ref, aw128_ref, awsc_ref,
               ep_ref, c2_ref, sc_ref, stats_ref, acc_ref):
    i = pl.program_id(0)
    ea = ea_ref[...]
    skv = skv_ref[...]
    k = skv[:, :D]
    v = skv[:, D:]
    s0 = k + dq_ref[...]
    ew = jnp.dot(ea, eww_ref[...], preferred_element_type=_F32) + ebw_ref[...]
    eb = jnp.dot(ea, ewb_ref[...], preferred_element_type=_F32) + ebb_ref[...]
    sc = s0 * ew
    sc = jnp.sqrt(jnp.maximum(sc, 0.0)) - jnp.sqrt(jnp.maximum(-sc, 0.0))
    et = jnp.maximum(sc + eb, 0.0)
    ep = jnp.dot(et, oew_ref[...], preferred_element_type=_F32) + oeb_ref[...] + ea
    ep_ref[...] = ep.astype(jnp.bfloat16)
    s128 = jnp.clip(jnp.dot(et, aw128_ref[...], preferred_element_type=_F32), -5.0, 5.0)
    p128 = jnp.exp(s128 - 5.0)
    if SCW == D:
        sc_ref[...] = p128
    else:
        ssc = jnp.clip(jnp.dot(et, awsc_ref[...], preferred_element_type=_F32),
                       -5.0, 5.0)
        sc_ref[...] = jnp.exp(ssc - 5.0)
    c2_ref[0] = p128 * v
    c2_ref[1] = p128 * et

    @pl.when(i == 0)
    def _():
        acc_ref[...] = jnp.zeros_like(acc_ref)

    acc_ref[0:1] = acc_ref[0:1] + jnp.sum(ep, axis=0, keepdims=True)
    acc_ref[1:2] = acc_ref[1:2] + jnp.sum(ep * ep, axis=0, keepdims=True)

    @pl.when(i == pl.num_programs(0) - 1)
    def _():
        stats_ref[...] = acc_ref[...]


def _bn_apply(x, stats_ref, g_ref, b_ref, count):
    mu = stats_ref[0:1] * (1.0 / count)
    var = stats_ref[1:2] * (1.0 / count) - mu * mu
    inv = lax.rsqrt(var + 1e-5)
    return g_ref[...] * (x - mu) * inv + b_ref[...]


def _bne_body(ep_ref, stats_ref, g_ref, b_ref, out_ref):
    out_ref[...] = _bn_apply(ep_ref[...].astype(_F32), stats_ref, g_ref, b_ref, E)


def _f1_body(wv_ref, rv_ref, ssa_ref, ssb_ref, x_ref, ld_ref, exp_ref, vebd_ref,
             dc0_ref, dc1_ref, ohw_ref, ohb_ref, hpre_ref, stats_ref, acc_ref):
    i = pl.program_id(0)
    ssr = jnp.dot(ssa_ref[0] + ssb_ref[0], exp_ref[...],
                  preferred_element_type=_F32)
    inv = 1.0 / (ssr + 1e-16)
    wv = wv_ref[0] * inv
    rv = jnp.dot(rv_ref[0] * inv, vebd_ref[...], preferred_element_type=_F32)
    wv = wv + rv
    hc = wv * (dc0_ref[...] + ld_ref[...] * dc1_ref[...])
    hp = jnp.dot(hc, ohw_ref[...], preferred_element_type=_F32) + ohb_ref[...] + x_ref[...]
    hpre_ref[...] = hp

    @pl.when(i == 0)
    def _():
        acc_ref[...] = jnp.zeros_like(acc_ref)

    acc_ref[0:1] = acc_ref[0:1] + jnp.sum(hp, axis=0, keepdims=True)
    acc_ref[1:2] = acc_ref[1:2] + jnp.sum(hp * hp, axis=0, keepdims=True)

    @pl.when(i == pl.num_programs(0) - 1)
    def _():
        stats_ref[...] = acc_ref[...]


def _f2_body(hp_ref, stats_ref, g1_ref, b1_ref, f1w_ref, f1b_ref,
             f2w_ref, f2b_ref, t_ref, stats2_ref, acc_ref):
    i = pl.program_id(0)
    h1 = _bn_apply(hp_ref[...], stats_ref, g1_ref, b1_ref, N)
    ffn = jnp.maximum(jnp.dot(h1, f1w_ref[...], preferred_element_type=_F32) + f1b_ref[...], 0.0)
    t = h1 + jnp.dot(ffn, f2w_ref[...], preferred_element_type=_F32) + f2b_ref[...]
    t_ref[...] = t

    @pl.when(i == 0)
    def _():
        acc_ref[...] = jnp.zeros_like(acc_ref)

    acc_ref[0:1] = acc_ref[0:1] + jnp.sum(t, axis=0, keepdims=True)
    acc_ref[1:2] = acc_ref[1:2] + jnp.sum(t * t, axis=0, keepdims=True)

    @pl.when(i == pl.num_programs(0) - 1)
    def _():
        stats2_ref[...] = acc_ref[...]


def _f3_body(t_ref, stats_ref, g_ref, b_ref, out_ref):
    out_ref[...] = _bn_apply(t_ref[...], stats_ref, g_ref, b_ref, N)


def _row(v):
    return v.reshape(1, -1)


def _wspec(shape):
    nd = len(shape)
    return pl.BlockSpec(shape, lambda i: (0,) * nd)


_ARB = pltpu.CompilerParams(dimension_semantics=("arbitrary",))


def kernel(x, edge_index, edge_attr, log_deg, num_nodes, Qw, Qb, Kw, Kb, Ew, Eb,
           Vw, Vb, Aw, VeRow, Ohw, Ohb, Oew, Oeb, deg_coef, bn1h_g, bn1h_b,
           bn1e_g, bn1e_b, bn2h_g, bn2h_b, f1w, f1b, f2w, f2b):
    perm = np.arange(D)
    hh, dd = perm // DH, perm % DH
    perm_w = jnp.asarray(hh * 2 * DH + dd)
    perm_b = jnp.asarray(hh * 2 * DH + DH + dd)
    Ew_w = Ew[:, perm_w]
    Ew_b = Ew[:, perm_b]
    eb_w = _row(Eb[perm_w])
    eb_b = _row(Eb[perm_b])
    A2 = Aw[:, :, 0]
    aw128 = jax.scipy.linalg.block_diag(
        *[jnp.broadcast_to(A2[:, h:h + 1], (DH, DH)) for h in range(H)])
    awsc = jax.scipy.linalg.block_diag(
        *[jnp.broadcast_to(A2[:, h:h + 1], (DH, SCW // H)) for h in range(H)])
    expm = jnp.asarray(np.kron(np.eye(H, dtype=np.float32),
                               np.full((SCW // H, DH), 1.0 / (SCW // H),
                                       np.float32)))
    vebd = jax.scipy.linalg.block_diag(*[VeRow[:, h, :] for h in range(H)])
    dc0 = _row(deg_coef[0, :, 0])
    dc1 = _row(deg_coef[0, :, 1])
    wkv = jnp.concatenate([Kw, Vw], axis=1)
    bkv = _row(jnp.concatenate([Kb, Vb]))
    src3w = edge_index[0].reshape(NW, (E // NW) // GC, GC)
    dst3w = edge_index[1].reshape(NW, (E // NW) // GC, GC)
    dst4t = edge_index[1].reshape(NS, 2, ((E // NS) // GC) // 2, GC)
    dst4c = edge_index[1].reshape(2, NS, ((E // 2) // NS) // GC, GC)
    ld128 = jnp.broadcast_to(log_deg, (N, D))

    kv_tab, q_tab = pl.pallas_call(
        _nodeproj_body,
        grid=(N // NB,),
        in_specs=[
            pl.BlockSpec((NB, D), lambda i: (i, 0)),
            _wspec((D, 2 * D)), _wspec((1, 2 * D)),
            _wspec((D, D)), _wspec((1, D)),
        ],
        out_specs=[
            pl.BlockSpec((NB, 2 * D), lambda i: (i, 0)),
            pl.BlockSpec((NB, D), lambda i: (i, 0)),
        ],
        out_shape=[
            jax.ShapeDtypeStruct((N, 2 * D), _F32),
            jax.ShapeDtypeStruct((N, D), _F32),
        ],
        compiler_params=_ARB,
    )(x, wkv, bkv, Qw, _row(Qb))

    skv = _make_gather(2 * D)(kv_tab, src3w)
    dq = _make_gather(D)(q_tab, dst3w)

    ep, c2, scw, stats_e = pl.pallas_call(
        _edge_body,
        grid=(E // EB,),
        in_specs=[
            pl.BlockSpec((EB, D), lambda i: (i, 0)),
            pl.BlockSpec((EB, 2 * D), lambda i: (i, 0)),
            pl.BlockSpec((EB, D), lambda i: (i, 0)),
            _wspec((D, D)), _wspec((D, D)), _wspec((1, D)), _wspec((1, D)),
            _wspec((D, D)), _wspec((1, D)), _wspec((D, D)), _wspec((D, SCW)),
        ],
        out_specs=[
            pl.BlockSpec((EB, D), lambda i: (i, 0)),
            pl.BlockSpec((2, EB, D), lambda i: (0, i, 0)),
            pl.BlockSpec((EB, SCW), lambda i: (i, 0)),
            pl.BlockSpec((8, D), lambda i: (0, 0)),
        ],
        out_shape=[
            jax.ShapeDtypeStruct((E, D), jnp.bfloat16),
            jax.ShapeDtypeStruct((2, E, D), _F32),
            jax.ShapeDtypeStruct((E, SCW), _F32),
            jax.ShapeDtypeStruct((8, D), _F32),
        ],
        scratch_shapes=[pltpu.VMEM((8, D), _F32)],
        compiler_params=_ARB,
    )(edge_attr, skv, dq, Ew_w, Ew_b, eb_w, eb_b, Oew, _row(Oeb), aw128, awsc)

    acc2 = _make_scatter2()(c2, dst4t)
    ss2 = _make_scatter_ss()(scw, dst4c)

    e_out = pl.pallas_call(
        _bne_body,
        grid=(E // EBN,),
        in_specs=[
            pl.BlockSpec((EBN, D), lambda i: (i, 0)),
            _wspec((8, D)), _wspec((1, D)), _wspec((1, D)),
        ],
        out_specs=pl.BlockSpec((EBN, D), lambda i: (i, 0)),
        out_shape=jax.ShapeDtypeStruct((E, D), _F32),
        compiler_params=_ARB,
    )(ep, stats_e, _row(bn1e_g), _row(bn1e_b))

    hpre, stats1 = pl.pallas_call(
        _f1_body,
        grid=(N // NB,),
        in_specs=[
            pl.BlockSpec((1, NB, D), lambda i: (0, i, 0)),
            pl.BlockSpec((1, NB, D), lambda i: (1, i, 0)),
            pl.BlockSpec((1, NB, SCW), lambda i: (0, i, 0)),
            pl.BlockSpec((1, NB, SCW), lambda i: (1, i, 0)),
            pl.BlockSpec((NB, D), lambda i: (i, 0)),
            pl.BlockSpec((NB, D), lambda i: (i, 0)),
            _wspec((SCW, D)), _wspec((D, D)),
            _wspec((1, D)), _wspec((1, D)), _wspec((D, D)), _wspec((1, D)),
        ],
        out_specs=[
            pl.BlockSpec((NB, D), lambda i: (i, 0)),
            pl.BlockSpec((8, D), lambda i: (0, 0)),
        ],
        out_shape=[
            jax.ShapeDtypeStruct((N, D), _F32),
            jax.ShapeDtypeStruct((8, D), _F32),
        ],
        scratch_shapes=[pltpu.VMEM((8, D), _F32)],
        compiler_params=_ARB,
    )(acc2, acc2, ss2, ss2, x, ld128, expm, vebd, dc0, dc1, Ohw, _row(Ohb))

    t_arr, stats2 = pl.pallas_call(
        _f2_body,
        grid=(N // NB,),
        in_specs=[
            pl.BlockSpec((NB, D), lambda i: (i, 0)),
            _wspec((8, D)), _wspec((1, D)), _wspec((1, D)),
            _wspec((D, 2 * D)), _wspec((1, 2 * D)),
            _wspec((2 * D, D)), _wspec((1, D)),
        ],
        out_specs=[
            pl.BlockSpec((NB, D), lambda i: (i, 0)),
            pl.BlockSpec((8, D), lambda i: (0, 0)),
        ],
        out_shape=[
            jax.ShapeDtypeStruct((N, D), _F32),
            jax.ShapeDtypeStruct((8, D), _F32),
        ],
        scratch_shapes=[pltpu.VMEM((8, D), _F32)],
        compiler_params=_ARB,
    )(hpre, stats1, _row(bn1h_g), _row(bn1h_b), f1w, _row(f1b), f2w, _row(f2b))

    h_out = pl.pallas_call(
        _f3_body,
        grid=(N // NB,),
        in_specs=[
            pl.BlockSpec((NB, D), lambda i: (i, 0)),
            _wspec((8, D)), _wspec((1, D)), _wspec((1, D)),
        ],
        out_specs=pl.BlockSpec((NB, D), lambda i: (i, 0)),
        out_shape=jax.ShapeDtypeStruct((N, D), _F32),
        compiler_params=_ARB,
    )(t_arr, stats2, _row(bn2h_g), _row(bn2h_b))

    return (h_out, e_out)


# speedup vs baseline: 1.1297x; 1.0322x over previous
import functools

import jax
import jax.numpy as jnp
import numpy as np
from jax import lax
from jax.experimental import pallas as pl
from jax.experimental.pallas import tpu as pltpu
from jax.experimental.pallas import tpu_sc as plsc

N = 10000
E = 320000
D = 128
H = 8
DH = D // H

NC = 2
NS = 16
NW = NC * NS
NPAD = 10240

SCW = 128
GC = 80
EB = 4000
NB = 2000
EBN = 2000

_F32 = jnp.float32


def _pipe2(n_iter, start_fn, wait_fn, consume_fn):
    start_fn(0, 0)
    start_fn(1, 1)

    def outer(g, _):
        for b in range(2):
            j = 2 * g + b
            wait_fn(j, b)
            consume_fn(j, b)

            @pl.when(j + 2 < n_iter)
            def _():
                start_fn(j + 2, b)

        return 0

    lax.fori_loop(0, (n_iter - 1) // 2, outer, 0, unroll=False)
    j = n_iter - 1
    wait_fn(j, j % 2)
    consume_fn(j, j % 2)


def _gather_body(table, idx3d_hbm, out, idx_v, buf0, buf1, sem0, sem1,
                 *, width, rows_per_w):
    c = lax.axis_index("c")
    s = lax.axis_index("s")
    wid = s * NC + c
    n_iter = rows_per_w // GC
    pltpu.sync_copy(idx3d_hbm.at[wid], idx_v)
    bufs = (buf0, buf1)
    sems = (sem0, sem1)

    def start(j, b):
        pltpu.async_copy(table.at[idx_v.at[j]], bufs[b], sems[b])

    def wait(j, b):
        pltpu.make_async_copy(table.at[idx_v.at[j]], bufs[b], sems[b]).wait()

    def consume(j, b):
        off = wid * rows_per_w + j * GC
        pltpu.sync_copy(bufs[b], out.at[pl.ds(off, GC)])

    _pipe2(n_iter, start, wait, consume)


def _make_gather(width):
    rows_per_w = E // NW
    mesh = plsc.VectorSubcoreMesh(core_axis_name="c", subcore_axis_name="s")
    return pl.kernel(
        functools.partial(_gather_body, width=width, rows_per_w=rows_per_w),
        out_type=jax.ShapeDtypeStruct((E, width), _F32),
        mesh=mesh,
        scratch_types=[
            pltpu.VMEM((rows_per_w // GC, GC), jnp.int32),
            pltpu.VMEM((GC, width), _F32),
            pltpu.VMEM((GC, width), _F32),
            pltpu.SemaphoreType.DMA,
            pltpu.SemaphoreType.DMA,
        ],
    )


def _zero_buf(b128):
    zv = jnp.zeros((16,), _F32)
    width = b128.shape[1]

    def zrow(r, _):
        for kk in range(width // 16):
            b128[r, pl.ds(kk * 16, 16)] = zv
        return 0

    lax.fori_loop(0, GC, zrow, 0, unroll=False)


def _zero_table(t128, b128, r0, nchunk):
    def zcp(k, _):
        pltpu.sync_copy(b128, t128.at[pl.ds(r0 + k * GC, GC)])
        return 0

    lax.fori_loop(0, nchunk, zcp, 0, unroll=False)


def _writeback(t128, b128, out_slice_fn, r0, nchunk):
    def wb(k, _):
        rr = r0 + k * GC
        pltpu.sync_copy(t128.at[pl.ds(rr, GC)], b128)
        pltpu.sync_copy(b128, out_slice_fn(rr))
        return 0

    lax.fori_loop(0, nchunk, wb, 0, unroll=False)


def _scatter2_body(c2_hbm, idx4d_hbm, acc2_out, t128, idx_v, b0, b1,
                   sem0, sem1):
    c = lax.axis_index("c")
    s = lax.axis_index("s")
    nrow = NPAD // NS
    r0 = s * nrow
    _zero_buf(b0)
    _zero_table(t128, b0, r0, nrow // GC)
    plsc.subcore_barrier()

    bufs = (b0, b1)
    sems = (sem0, sem1)
    rows_per_t = E // NS
    n_half = (rows_per_t // GC) // 2
    for half in range(2):
        pltpu.sync_copy(idx4d_hbm.at[s, half], idx_v)

        def start(j, b, half=half):
            off = s * rows_per_t + (half * n_half + j) * GC
            pltpu.async_copy(c2_hbm.at[c, pl.ds(off, GC)], bufs[b], sems[b])

        def wait(j, b, half=half):
            off = s * rows_per_t + (half * n_half + j) * GC
            pltpu.make_async_copy(c2_hbm.at[c, pl.ds(off, GC)], bufs[b],
                                  sems[b]).wait()

        def consume(j, b):
            pltpu.sync_copy(bufs[b], t128.at[idx_v.at[j]], add=True)

        _pipe2(n_half, start, wait, consume)
    plsc.subcore_barrier()
    _writeback(t128, b0, lambda rr: acc2_out.at[c, pl.ds(rr, GC)],
               r0, nrow // GC)


def _make_scatter2():
    mesh = plsc.VectorSubcoreMesh(core_axis_name="c", subcore_axis_name="s")
    n_half = ((E // NS) // GC) // 2
    return pl.kernel(
        _scatter2_body,
        out_type=jax.ShapeDtypeStruct((2, NPAD, D), _F32),
        mesh=mesh,
        scratch_types=[
            pltpu.MemorySpace.VMEM_SHARED((NPAD, D), _F32),
            pltpu.VMEM((n_half, GC), jnp.int32),
            pltpu.VMEM((GC, D), _F32),
            pltpu.VMEM((GC, D), _F32),
            pltpu.SemaphoreType.DMA,
            pltpu.SemaphoreType.DMA,
        ],
    )


def _scatter_ss_body(sc_hbm, idx4d_hbm, ss2_out, t128, idx_v, b0, b1,
                     sem0, sem1):
    c = lax.axis_index("c")
    s = lax.axis_index("s")
    nrow = NPAD // NS
    r0 = s * nrow
    _zero_buf(b0)
    _zero_table(t128, b0, r0, nrow // GC)
    plsc.subcore_barrier()

    bufs = (b0, b1)
    sems = (sem0, sem1)
    rows_per_t = (E // 2) // NS
    n_iter = rows_per_t // GC
    pltpu.sync_copy(idx4d_hbm.at[c, s], idx_v)

    def start(j, b):
        off = c * (E // 2) + s * rows_per_t + j * GC
        pltpu.async_copy(sc_hbm.at[pl.ds(off, GC)], bufs[b], sems[b])

    def wait(j, b):
        off = c * (E // 2) + s * rows_per_t + j * GC
        pltpu.make_async_copy(sc_hbm.at[pl.ds(off, GC)], bufs[b],
                              sems[b]).wait()

    def consume(j, b):
        pltpu.sync_copy(bufs[b], t128.at[idx_v.at[j]], add=True)

    _pipe2(n_iter, start, wait, consume)
    plsc.subcore_barrier()
    _writeback(t128, b0, lambda rr: ss2_out.at[c, pl.ds(rr, GC)],
               r0, nrow // GC)


def _make_scatter_ss():
    mesh = plsc.VectorSubcoreMesh(core_axis_name="c", subcore_axis_name="s")
    n_iter = ((E // 2) // NS) // GC
    return pl.kernel(
        _scatter_ss_body,
        out_type=jax.ShapeDtypeStruct((2, NPAD, SCW), _F32),
        mesh=mesh,
        scratch_types=[
            pltpu.MemorySpace.VMEM_SHARED((NPAD, SCW), _F32),
            pltpu.VMEM((n_iter, GC), jnp.int32),
            pltpu.VMEM((GC, SCW), _F32),
            pltpu.VMEM((GC, SCW), _F32),
            pltpu.SemaphoreType.DMA,
            pltpu.SemaphoreType.DMA,
        ],
    )


def _nodeproj_body(x_ref, wkv_ref, bkv_ref, wq_ref, bq_ref, kv_ref, q_ref):
    x = x_ref[...]
    kv_ref[...] = jnp.dot(x, wkv_ref[...], preferred_element_type=_F32) + bkv_ref[...]
    q_ref[...] = jnp.dot(x, wq_ref[...], preferred_element_type=_F32) + bq_ref[...]


def _edge_body(ea_ref, skv_ref, dq_ref, eww_ref, ewb_ref, ebw_ref, ebb_ref,
               oew_ref, oeb_ref, aw128_ref, awsc_ref,
               ep_ref, c2_ref, sc_ref, stats_ref, acc_ref):
    i = pl.program_id(0)
    ea = ea_ref[...]
    skv = skv_ref[...]
    k = skv[:, :D]
    v = skv[:, D:]
    s0 = k + dq_ref[...]
    ew = jnp.dot(ea, eww_ref[...], preferred_element_type=_F32) + ebw_ref[...]
    eb = jnp.dot(ea, ewb_ref[...], preferred_element_type=_F32) + ebb_ref[...]
    sc = s0 * ew
    sc = jnp.sqrt(jnp.maximum(sc, 0.0)) - jnp.sqrt(jnp.maximum(-sc, 0.0))
    et = jnp.maximum(sc + eb, 0.0)
    ep = jnp.dot(et, oew_ref[...], preferred_element_type=_F32) + oeb_ref[...] + ea
    ep_ref[...] = ep.astype(jnp.bfloat16)
    s128 = jnp.clip(jnp.dot(et, aw128_ref[...], preferred_element_type=_F32), -5.0, 5.0)
    p128 = jnp.exp(s128 - 5.0)
    if SCW == D:
        sc_ref[...] = p128
    else:
        ssc = jnp.clip(jnp.dot(et, awsc_ref[...], preferred_element_type=_F32),
                       -5.0, 5.0)
        sc_ref[...] = jnp.exp(ssc - 5.0)
    c2_ref[0] = p128 * v
    c2_ref[1] = p128 * et

    @pl.when(i == 0)
    def _():
        acc_ref[...] = jnp.zeros_like(acc_ref)

    acc_ref[0:1] = acc_ref[0:1] + jnp.sum(ep, axis=0, keepdims=True)
    acc_ref[1:2] = acc_ref[1:2] + jnp.sum(ep * ep, axis=0, keepdims=True)

    @pl.when(i == pl.num_programs(0) - 1)
    def _():
        stats_ref[...] = acc_ref[...]


def _bn_apply(x, stats_ref, g_ref, b_ref, count):
    mu = stats_ref[0:1] * (1.0 / count)
    var = stats_ref[1:2] * (1.0 / count) - mu * mu
    inv = lax.rsqrt(var + 1e-5)
    return g_ref[...] * (x - mu) * inv + b_ref[...]


def _bne_body(ep_ref, stats_ref, g_ref, b_ref, out_ref):
    out_ref[...] = _bn_apply(ep_ref[...].astype(_F32), stats_ref, g_ref, b_ref, E)


def _f1_body(wv_ref, rv_ref, ssa_ref, ssb_ref, x_ref, ld_ref, exp_ref, vebd_ref,
             dc0_ref, dc1_ref, ohw_ref, ohb_ref, hpre_ref, stats_ref, acc_ref):
    i = pl.program_id(0)
    ssr = jnp.dot(ssa_ref[0] + ssb_ref[0], exp_ref[...],
                  preferred_element_type=_F32)
    inv = 1.0 / (ssr + 1e-16)
    wv = wv_ref[0] * inv
    rv = jnp.dot(rv_ref[0] * inv, vebd_ref[...], preferred_element_type=_F32)
    wv = wv + rv
    hc = wv * (dc0_ref[...] + ld_ref[...] * dc1_ref[...])
    hp = jnp.dot(hc, ohw_ref[...], preferred_element_type=_F32) + ohb_ref[...] + x_ref[...]
    hpre_ref[...] = hp

    @pl.when(i == 0)
    def _():
        acc_ref[...] = jnp.zeros_like(acc_ref)

    acc_ref[0:1] = acc_ref[0:1] + jnp.sum(hp, axis=0, keepdims=True)
    acc_ref[1:2] = acc_ref[1:2] + jnp.sum(hp * hp, axis=0, keepdims=True)

    @pl.when(i == pl.num_programs(0) - 1)
    def _():
        stats_ref[...] = acc_ref[...]


def _f2_body(hp_ref, stats_ref, g1_ref, b1_ref, f1w_ref, f1b_ref,
             f2w_ref, f2b_ref, t_ref, stats2_ref, acc_ref):
    i = pl.program_id(0)
    h1 = _bn_apply(hp_ref[...], stats_ref, g1_ref, b1_ref, N)
    ffn = jnp.maximum(jnp.dot(h1, f1w_ref[...], preferred_element_type=_F32) + f1b_ref[...], 0.0)
    t = h1 + jnp.dot(ffn, f2w_ref[...], preferred_element_type=_F32) + f2b_ref[...]
    t_ref[...] = t

    @pl.when(i == 0)
    def _():
        acc_ref[...] = jnp.zeros_like(acc_ref)

    acc_ref[0:1] = acc_ref[0:1] + jnp.sum(t, axis=0, keepdims=True)
    acc_ref[1:2] = acc_ref[1:2] + jnp.sum(t * t, axis=0, keepdims=True)

    @pl.when(i == pl.num_programs(0) - 1)
    def _():
        stats2_ref[...] = acc_ref[...]


def _f3_body(t_ref, stats_ref, g_ref, b_ref, out_ref):
    out_ref[...] = _bn_apply(t_ref[...], stats_ref, g_ref, b_ref, N)


def _row(v):
    return v.reshape(1, -1)


def _wspec(shape):
    nd = len(shape)
    return pl.BlockSpec(shape, lambda i: (0,) * nd)


_ARB = pltpu.CompilerParams(dimension_semantics=("arbitrary",))


def kernel(x, edge_index, edge_attr, log_deg, num_nodes, Qw, Qb, Kw, Kb, Ew, Eb,
           Vw, Vb, Aw, VeRow, Ohw, Ohb, Oew, Oeb, deg_coef, bn1h_g, bn1h_b,
           bn1e_g, bn1e_b, bn2h_g, bn2h_b, f1w, f1b, f2w, f2b):
    perm = np.arange(D)
    hh, dd = perm // DH, perm % DH
    perm_w = jnp.asarray(hh * 2 * DH + dd)
    perm_b = jnp.asarray(hh * 2 * DH + DH + dd)
    Ew_w = Ew[:, perm_w]
    Ew_b = Ew[:, perm_b]
    eb_w = _row(Eb[perm_w])
    eb_b = _row(Eb[perm_b])
    A2 = Aw[:, :, 0]
    aw128 = jax.scipy.linalg.block_diag(
        *[jnp.broadcast_to(A2[:, h:h + 1], (DH, DH)) for h in range(H)])
    awsc = jax.scipy.linalg.block_diag(
        *[jnp.broadcast_to(A2[:, h:h + 1], (DH, SCW // H)) for h in range(H)])
    expm = jnp.asarray(np.kron(np.eye(H, dtype=np.float32),
                               np.full((SCW // H, DH), 1.0 / (SCW // H),
                                       np.float32)))
    vebd = jax.scipy.linalg.block_diag(*[VeRow[:, h, :] for h in range(H)])
    dc0 = _row(deg_coef[0, :, 0])
    dc1 = _row(deg_coef[0, :, 1])
    wkv = jnp.concatenate([Kw, Vw], axis=1)
    bkv = _row(jnp.concatenate([Kb, Vb]))
    src3w = edge_index[0].reshape(NW, (E // NW) // GC, GC)
    dst3w = edge_index[1].reshape(NW, (E // NW) // GC, GC)
    dst4t = edge_index[1].reshape(NS, 2, ((E // NS) // GC) // 2, GC)
    dst4c = edge_index[1].reshape(2, NS, ((E // 2) // NS) // GC, GC)
    ld128 = jnp.broadcast_to(log_deg, (N, D))

    kv_tab, q_tab = pl.pallas_call(
        _nodeproj_body,
        grid=(N // NB,),
        in_specs=[
            pl.BlockSpec((NB, D), lambda i: (i, 0)),
            _wspec((D, 2 * D)), _wspec((1, 2 * D)),
            _wspec((D, D)), _wspec((1, D)),
        ],
        out_specs=[
            pl.BlockSpec((NB, 2 * D), lambda i: (i, 0)),
            pl.BlockSpec((NB, D), lambda i: (i, 0)),
        ],
        out_shape=[
            jax.ShapeDtypeStruct((N, 2 * D), _F32),
            jax.ShapeDtypeStruct((N, D), _F32),
        ],
        compiler_params=_ARB,
    )(x, wkv, bkv, Qw, _row(Qb))

    skv = _make_gather(2 * D)(kv_tab, src3w)
    dq = _make_gather(D)(q_tab, dst3w)

    ep, c2, scw, stats_e = pl.pallas_call(
        _edge_body,
        grid=(E // EB,),
        in_specs=[
            pl.BlockSpec((EB, D), lambda i: (i, 0)),
            pl.BlockSpec((EB, 2 * D), lambda i: (i, 0)),
            pl.BlockSpec((EB, D), lambda i: (i, 0)),
            _wspec((D, D)), _wspec((D, D)), _wspec((1, D)), _wspec((1, D)),
            _wspec((D, D)), _wspec((1, D)), _wspec((D, D)), _wspec((D, SCW)),
        ],
        out_specs=[
            pl.BlockSpec((EB, D), lambda i: (i, 0)),
            pl.BlockSpec((2, EB, D), lambda i: (0, i, 0)),
            pl.BlockSpec((EB, SCW), lambda i: (i, 0)),
            pl.BlockSpec((8, D), lambda i: (0, 0)),
        ],
        out_shape=[
            jax.ShapeDtypeStruct((E, D), jnp.bfloat16),
            jax.ShapeDtypeStruct((2, E, D), _F32),
            jax.ShapeDtypeStruct((E, SCW), _F32),
            jax.ShapeDtypeStruct((8, D), _F32),
        ],
        scratch_shapes=[pltpu.VMEM((8, D), _F32)],
        compiler_params=_ARB,
    )(edge_attr, skv, dq, Ew_w, Ew_b, eb_w, eb_b, Oew, _row(Oeb), aw128, awsc)

    acc2 = _make_scatter2()(c2, dst4t)
    ss2 = _make_scatter_ss()(scw, dst4c)

    e_out = pl.pallas_call(
        _bne_body,
        grid=(E // EBN,),
        in_specs=[
            pl.BlockSpec((EBN, D), lambda i: (i, 0)),
            _wspec((8, D)), _wspec((1, D)), _wspec((1, D)),
        ],
        out_specs=pl.BlockSpec((EBN, D), lambda i: (i, 0)),
        out_shape=jax.ShapeDtypeStruct((E, D), _F32),
        compiler_params=_ARB,
    )(ep, stats_e, _row(bn1e_g), _row(bn1e_b))

    hpre, stats1 = pl.pallas_call(
        _f1_body,
        grid=(N // NB,),
        in_specs=[
            pl.BlockSpec((1, NB, D), lambda i: (0, i, 0)),
            pl.BlockSpec((1, NB, D), lambda i: (1, i, 0)),
            pl.BlockSpec((1, NB, SCW), lambda i: (0, i, 0)),
            pl.BlockSpec((1, NB, SCW), lambda i: (1, i, 0)),
            pl.BlockSpec((NB, D), lambda i: (i, 0)),
            pl.BlockSpec((NB, D), lambda i: (i, 0)),
            _wspec((SCW, D)), _wspec((D, D)),
            _wspec((1, D)), _wspec((1, D)), _wspec((D, D)), _wspec((1, D)),
        ],
        out_specs=[
            pl.BlockSpec((NB, D), lambda i: (i, 0)),
            pl.BlockSpec((8, D), lambda i: (0, 0)),
        ],
        out_shape=[
            jax.ShapeDtypeStruct((N, D), _F32),
            jax.ShapeDtypeStruct((8, D), _F32),
        ],
        scratch_shapes=[pltpu.VMEM((8, D), _F32)],
        compiler_params=_ARB,
    )(acc2, acc2, ss2, ss2, x, ld128, expm, vebd, dc0, dc1, Ohw, _row(Ohb))

    t_arr, stats2 = pl.pallas_call(
        _f2_body,
        grid=(N // NB,),
        in_specs=[
            pl.BlockSpec((NB, D), lambda i: (i, 0)),
            _wspec((8, D)), _wspec((1, D)), _wspec((1, D)),
            _wspec((D, 2 * D)), _wspec((1, 2 * D)),
            _wspec((2 * D, D)), _wspec((1, D)),
        ],
        out_specs=[
            pl.BlockSpec((NB, D), lambda i: (i, 0)),
            pl.BlockSpec((8, D), lambda i: (0, 0)),
        ],
        out_shape=[
            jax.ShapeDtypeStruct((N, D), _F32),
            jax.ShapeDtypeStruct((8, D), _F32),
        ],
        scratch_shapes=[pltpu.VMEM((8, D), _F32)],
        compiler_params=_ARB,
    )(hpre, stats1, _row(bn1h_g), _row(bn1h_b), f1w, _row(f1b), f2w, _row(f2b))

    h_out = pl.pallas_call(
        _f3_body,
        grid=(N // NB,),
        in_specs=[
            pl.BlockSpec((NB, D), lambda i: (i, 0)),
            _wspec((8, D)), _wspec((1, D)), _wspec((1, D)),
        ],
        out_specs=pl.BlockSpec((NB, D), lambda i: (i, 0)),
        out_shape=jax.ShapeDtypeStruct((N, D), _F32),
        compiler_params=_ARB,
    )(t_arr, stats2, _row(bn2h_g), _row(bn2h_b))

    return (h_out, e_out)
